# baseline (speedup 1.0000x reference)
import functools

import jax
import jax.numpy as jnp
import numpy as np
from jax import lax
from jax.experimental import pallas as pl
from jax.experimental.pallas import tpu as pltpu
from jax.experimental.pallas import tpu_sc as plsc

BATCH = 1024
D_IN = 4096
D_OUT = 8192
NWORKERS = 32
ROWS_PER_W = BATCH // NWORKERS
XR = 8
NCHUNK = ROWS_PER_W // XR
JC = 1024
NJC = D_OUT // JC
L = 16

_GATE_COEF = np.array(
    [
        [0, 0, 0, 0, 0, 0, 0, 0, 1, 1, 1, 1, 1, 1, 1, 1],
        [0, 0, 1, 1, 0, 0, 1, 1, -1, -1, 0, 0, -1, -1, 0, 0],
        [0, 0, 0, 0, 1, 1, 1, 1, -1, -1, -1, -1, 0, 0, 0, 0],
        [0, 1, -1, 0, -1, 0, -2, -1, 1, 2, 0, 1, 0, 1, -1, 0],
    ],
    dtype=np.float32,
)


JW = D_OUT // 16


def _sc_body(x_hbm, wt_hbm, iaib_hbm, out_hbm,
             xb, ob0, ob1, iaibb, cfb, wvm, cfsh, sem_x, sem_o0, sem_o1):
    wid = lax.axis_index("s") * 2 + lax.axis_index("c")
    row0 = wid * ROWS_PER_W

    def x_copies(c):
        return [
            pltpu.make_async_copy(
                x_hbm.at[row0 + c * XR + n],
                xb.at[pl.ds(((c % 2) * XR + n) * D_IN, D_IN)],
                sem_x,
            )
            for n in range(XR)
        ]

    def out_copies(c, jcb, bo):
        sem = sem_o0 if bo == 0 else sem_o1
        ob = ob0 if bo == 0 else ob1
        return [
            pltpu.make_async_copy(
                ob,
                out_hbm.at[pl.ds(row0 + c * XR, XR), pl.ds(jcb * JC, JC)],
                sem,
            )
        ]

    for cp in x_copies(0):
        cp.start()
    pltpu.sync_copy(iaib_hbm, iaibb)

    sid = lax.axis_index("s")
    j0 = sid * JW
    for k in range(16):
        pltpu.sync_copy(wt_hbm.at[k, pl.ds(j0, JW)],
                        wvm.at[pl.ds(k * JW, JW)])

    @plsc.parallel_loop(0, JW // L, unroll=1)
    def coef_blk(tl):
        jl = tl * L
        w = [wvm[pl.ds(k * JW + jl, L)] for k in range(16)]
        m = w[0]
        for k in range(1, 16):
            m = jnp.maximum(m, w[k])
        e = [jnp.exp(wk - m) for wk in w]
        s = e[0]
        for k in range(1, 16):
            s = s + e[k]
        r = 1.0 / s
        p = [ek * r for ek in e]
        c0 = p[8] + p[9] + p[10] + p[11] + p[12] + p[13] + p[14] + p[15]
        ca = p[2] + p[3] + p[6] + p[7] - p[8] - p[9] - p[12] - p[13]
        cb = p[4] + p[5] + p[6] + p[7] - p[8] - p[9] - p[10] - p[11]
        cab = (p[1] - p[2] - p[4] - 2.0 * p[6] - p[7] + p[8]
               + 2.0 * p[9] + p[11] + p[13] - p[14])
        w0 = plsc.bitcast(
            plsc.pack(c0, ca, format=plsc.PackFormat.INTERLEAVED), jnp.int32)
        w1 = plsc.bitcast(
            plsc.pack(cb, cab, format=plsc.PackFormat.INTERLEAVED), jnp.int32)
        cfb[pl.ds(j0 + jl, L)] = w0
        cfb[pl.ds(D_OUT + j0 + jl, L)] = w1

    pltpu.sync_copy(cfb.at[pl.ds(j0, JW)], cfsh.at[pl.ds(j0, JW)])
    pltpu.sync_copy(cfb.at[pl.ds(D_OUT + j0, JW)],
                    cfsh.at[pl.ds(D_OUT + j0, JW)])
    plsc.subcore_barrier()
    pltpu.sync_copy(cfsh, cfb)

    def sub_body(c, jcb, bo, drain):
        if drain:
            for cp in out_copies(c, jcb, bo):
                cp.wait()

        ob = ob0 if bo == 0 else ob1

        @plsc.parallel_loop(0, JC // L, unroll=2)
        def jblk(t):
            col = jcb * JC + t * L
            iaib = iaibb[pl.ds(col, L)]
            ia = iaib & 0xFFFF
            ib = iaib >> 16
            w0 = cfb[pl.ds(col, L)]
            w1 = cfb[pl.ds(col + D_OUT, L)]
            c0 = plsc.bitcast(w0 << 16, jnp.float32)
            ca = plsc.bitcast(w0 & -65536, jnp.float32)
            cb = plsc.bitcast(w1 << 16, jnp.float32)
            cab = plsc.bitcast(w1 & -65536, jnp.float32)
            for n in range(XR):
                xrow = xb.at[pl.ds(((c % 2) * XR + n) * D_IN, D_IN)]
                a = plsc.load_gather(xrow, [ia])
                b = plsc.load_gather(xrow, [ib])
                ob[n, pl.ds(t * L, L)] = (c0 + ca * a) + b * (cb + cab * a)

        for cp in out_copies(c, jcb, bo):
            cp.start()

    for c in range(NCHUNK):
        for cp in x_copies(c):
            cp.wait()
        if c + 1 < NCHUNK:
            for cp in x_copies(c + 1):
                cp.start()
        if c == 0:
            sub_body(0, 0, 0, drain=False)
            sub_body(0, 1, 1, drain=False)

            def pair_body0(p, carry):
                sub_body(0, 2 * p, 0, drain=True)
                sub_body(0, 2 * p + 1, 1, drain=True)
                return carry

            lax.fori_loop(1, NJC // 2, pair_body0, 0)
        else:
            def pair_body(p, carry, c=c):
                sub_body(c, 2 * p, 0, drain=True)
                sub_body(c, 2 * p + 1, 1, drain=True)
                return carry

            lax.fori_loop(0, NJC // 2, pair_body, 0)

    for cp in out_copies(NCHUNK - 1, NJC - 2, 0):
        cp.wait()
    for cp in out_copies(NCHUNK - 1, NJC - 1, 1):
        cp.wait()


_sc_call = functools.partial(
    pl.kernel,
    mesh=plsc.VectorSubcoreMesh(core_axis_name="c", subcore_axis_name="s"),
    compiler_params=pltpu.CompilerParams(needs_layout_passes=False),
    out_type=jax.ShapeDtypeStruct((BATCH, D_OUT), jnp.float32),
    scratch_types=[
        pltpu.VMEM((2 * XR * D_IN,), jnp.float32),
        pltpu.VMEM((XR, JC), jnp.float32),
        pltpu.VMEM((XR, JC), jnp.float32),
        pltpu.VMEM((D_OUT,), jnp.int32),
        pltpu.VMEM((2 * D_OUT,), jnp.int32),
        pltpu.VMEM((16 * JW,), jnp.float32),
        pltpu.VMEM_SHARED((2 * D_OUT,), jnp.int32),
        pltpu.SemaphoreType.DMA,
        pltpu.SemaphoreType.DMA,
        pltpu.SemaphoreType.DMA,
    ],
)(_sc_body)


def kernel(x, weights, idx_a, idx_b):
    wt = weights.T.astype(jnp.float32)
    iaib = (idx_a.astype(jnp.int32)
            | (idx_b.astype(jnp.int32) << 16))
    return _sc_call(x, wt, iaib)


# speedup vs baseline: 1.1208x; 1.1208x over previous
import functools

import jax
import jax.numpy as jnp
import numpy as np
from jax import lax
from jax.experimental import pallas as pl
from jax.experimental.pallas import tpu as pltpu
from jax.experimental.pallas import tpu_sc as plsc

BATCH = 1024
D_IN = 4096
D_OUT = 8192
NWORKERS = 32
ROWS_PER_W = BATCH // NWORKERS
XR = 8
NCHUNK = ROWS_PER_W // XR
JC = 1024
NJC = D_OUT // JC
L = 16

_GATE_COEF = np.array(
    [
        [0, 0, 0, 0, 0, 0, 0, 0, 1, 1, 1, 1, 1, 1, 1, 1],
        [0, 0, 1, 1, 0, 0, 1, 1, -1, -1, 0, 0, -1, -1, 0, 0],
        [0, 0, 0, 0, 1, 1, 1, 1, -1, -1, -1, -1, 0, 0, 0, 0],
        [0, 1, -1, 0, -1, 0, -2, -1, 1, 2, 0, 1, 0, 1, -1, 0],
    ],
    dtype=np.float32,
)


def _coef_body(wt_ref, out_ref):
    wt = wt_ref[...]
    m = jnp.max(wt, axis=0, keepdims=True)
    e = jnp.exp(wt - m)
    p = e / jnp.sum(e, axis=0, keepdims=True)
    rows = []
    for k in range(4):
        row = jnp.zeros((1, D_OUT), jnp.float32)
        for i in range(16):
            g = float(_GATE_COEF[k, i])
            if g == 0.0:
                continue
            row = row + g * p[i : i + 1, :]
        rows.append(row)

    def pack(hi, lo):
        hb = jax.lax.bitcast_convert_type(
            hi.astype(jnp.bfloat16), jnp.uint16).astype(jnp.uint32)
        lb = jax.lax.bitcast_convert_type(
            lo.astype(jnp.bfloat16), jnp.uint16).astype(jnp.uint32)
        return ((hb << 16) | lb).astype(jnp.int32)

    c0, ca, cb, cab = rows
    out_ref[...] = jnp.concatenate([pack(ca, c0), pack(cab, cb)], axis=0)


_coef_call = pl.pallas_call(
    _coef_body,
    out_shape=jax.ShapeDtypeStruct((2, D_OUT), jnp.int32),
)


def _sc_body(x_hbm, cf_hbm, iaib_hbm, out_hbm,
             xb, ob0, ob1, iaibb, cfb, sem_x, sem_o0, sem_o1):
    wid = lax.axis_index("s") * 2 + lax.axis_index("c")
    row0 = wid * ROWS_PER_W

    def x_copies(c):
        return [
            pltpu.make_async_copy(
                x_hbm.at[row0 + c * XR + n],
                xb.at[pl.ds(((c % 2) * XR + n) * D_IN, D_IN)],
                sem_x,
            )
            for n in range(XR)
        ]

    def out_copies(c, jcb, bo):
        sem = sem_o0 if bo == 0 else sem_o1
        ob = ob0 if bo == 0 else ob1
        return [
            pltpu.make_async_copy(
                ob,
                out_hbm.at[pl.ds(row0 + c * XR, XR), pl.ds(jcb * JC, JC)],
                sem,
            )
        ]

    for cp in x_copies(0):
        cp.start()
    pltpu.sync_copy(iaib_hbm, iaibb)
    for k in range(2):
        pltpu.sync_copy(cf_hbm.at[k], cfb.at[pl.ds(k * D_OUT, D_OUT)])

    def sub_body(c, jcb, bo, drain):
        if drain:
            for cp in out_copies(c, jcb, bo):
                cp.wait()

        ob = ob0 if bo == 0 else ob1

        @plsc.parallel_loop(0, JC // L, unroll=2)
        def jblk(t):
            col = jcb * JC + t * L
            iaib = iaibb[pl.ds(col, L)]
            ia = iaib & 0xFFFF
            ib = iaib >> 16
            w0 = cfb[pl.ds(col, L)]
            w1 = cfb[pl.ds(col + D_OUT, L)]
            c0 = plsc.bitcast(w0 << 16, jnp.float32)
            ca = plsc.bitcast(w0 & -65536, jnp.float32)
            cb = plsc.bitcast(w1 << 16, jnp.float32)
            cab = plsc.bitcast(w1 & -65536, jnp.float32)
            for n in range(XR):
                xrow = xb.at[pl.ds(((c % 2) * XR + n) * D_IN, D_IN)]
                a = plsc.load_gather(xrow, [ia])
                b = plsc.load_gather(xrow, [ib])
                ob[n, pl.ds(t * L, L)] = (c0 + ca * a) + b * (cb + cab * a)

        for cp in out_copies(c, jcb, bo):
            cp.start()

    for c in range(NCHUNK):
        for cp in x_copies(c):
            cp.wait()
        if c + 1 < NCHUNK:
            for cp in x_copies(c + 1):
                cp.start()
        if c == 0:
            sub_body(0, 0, 0, drain=False)
            sub_body(0, 1, 1, drain=False)

            def pair_body0(p, carry):
                sub_body(0, 2 * p, 0, drain=True)
                sub_body(0, 2 * p + 1, 1, drain=True)
                return carry

            lax.fori_loop(1, NJC // 2, pair_body0, 0)
        else:
            def pair_body(p, carry, c=c):
                sub_body(c, 2 * p, 0, drain=True)
                sub_body(c, 2 * p + 1, 1, drain=True)
                return carry

            lax.fori_loop(0, NJC // 2, pair_body, 0)

    for cp in out_copies(NCHUNK - 1, NJC - 2, 0):
        cp.wait()
    for cp in out_copies(NCHUNK - 1, NJC - 1, 1):
        cp.wait()


_sc_call = functools.partial(
    pl.kernel,
    mesh=plsc.VectorSubcoreMesh(core_axis_name="c", subcore_axis_name="s"),
    compiler_params=pltpu.CompilerParams(needs_layout_passes=False),
    out_type=jax.ShapeDtypeStruct((BATCH, D_OUT), jnp.float32),
    scratch_types=[
        pltpu.VMEM((2 * XR * D_IN,), jnp.float32),
        pltpu.VMEM((XR, JC), jnp.float32),
        pltpu.VMEM((XR, JC), jnp.float32),
        pltpu.VMEM((D_OUT,), jnp.int32),
        pltpu.VMEM((2 * D_OUT,), jnp.int32),
        pltpu.SemaphoreType.DMA,
        pltpu.SemaphoreType.DMA,
        pltpu.SemaphoreType.DMA,
    ],
)(_sc_body)


def kernel(x, weights, idx_a, idx_b):
    wt = weights.T.astype(jnp.float32)
    cf = _coef_call(wt)
    iaib = (idx_a.astype(jnp.int32)
            | (idx_b.astype(jnp.int32) << 16))
    return _sc_call(x, cf, iaib)
